# baseline (speedup 1.0000x reference)
import functools

import jax
import jax.numpy as jnp
from jax import lax
from jax.experimental import pallas as pl
from jax.experimental.pallas import tpu as pltpu

BOUND_OFF = 3.0
_PREC = jax.lax.Precision.HIGHEST


def _stats_kernel(h_ref, t_ref, vb_ref, oh_ref, idxr_ref, bd_ref, *, J, V, G):
    b = pl.program_id(0)
    fmax = float(V - 1)
    n = V * V * V
    lanes = n // V
    pxs, pys, pzs = [], [], []
    r_i = lax.broadcasted_iota(jnp.int32, (V, lanes), 0)
    c_i = lax.broadcasted_iota(jnp.int32, (V, lanes), 1)
    ii = r_i.astype(jnp.float32)
    jj = (c_i // V).astype(jnp.float32)
    kk = (c_i % V).astype(jnp.float32)
    for j in range(J):
        hj = h_ref[0, j]
        mx = jnp.max(hj)
        mask = (hj == mx).astype(jnp.float32)
        cnt = jnp.sum(mask)
        pxs.append(jnp.sum(mask * ii) / cnt)
        pys.append(jnp.sum(mask * jj) / cnt)
        pzs.append(jnp.sum(mask * kk) / cnt)

    def _build(axis, plist):
        mn = functools.reduce(jnp.minimum, plist)
        mxp = functools.reduce(jnp.maximum, plist)
        max_bd = jnp.clip(mxp + BOUND_OFF, 0.0, fmax)
        min_bd = jnp.clip(mn - BOUND_OFF, 0.0, fmax)
        vmin_a = vb_ref[b, axis]
        vmax_a = vb_ref[b, 3 + axis]
        interval = (vmax_a - vmin_a) / fmax
        max_b = vmin_a + max_bd / fmax * (vmax_a - vmin_a)
        min_b = vmin_a + min_bd / fmax * (vmax_a - vmin_a)
        return min_b, max_b, vmin_a, interval

    mnx, mxx, vminx, intx = _build(0, pxs)
    mny, mxy, vminy, inty = _build(1, pys)
    mnz, mxz, vminz, intz = _build(2, pzs)

    t32 = jnp.broadcast_to(t_ref[0:1, :], (V, G))
    vlane = lax.broadcasted_iota(jnp.int32, (V, G), 0).astype(jnp.float32)
    for slot, (mn, mxw, vmin_a, inter) in enumerate(
        [(mny, mxy, vminy, inty), (mnz, mxz, vminz, intz)]
    ):
        g = mn + t32 * (mxw - mn)
        vox = (g - vmin_a) / inter
        idxf = jnp.clip(jnp.round(vox), 0.0, fmax)
        oh_ref[0, slot] = (idxf == vlane).astype(jnp.float32)

    gx = mnx + t_ref[...] * (mxx - mnx)
    voxx = (gx - vminx) / intx
    idxr_ref[0] = jnp.clip(jnp.round(voxx), 0.0, fmax)

    lane = lax.broadcasted_iota(jnp.int32, (8, 128), 1)
    bd = jnp.zeros((8, 128), jnp.float32)
    for p, val in enumerate([mnx, mny, mnz, mxx, mxy, mxz]):
        bd = jnp.where(lane == p, val, bd)
    bd_ref[0] = bd


def _expand_kernel(h_ref, oh_ref, idx_ref, out_ref, sc2_ref, a3_ref, *, V, G):
    b = pl.program_id(0)
    hv = h_ref[0, 0]
    ht = jnp.transpose(hv, (0, 2, 1))
    a = ht.reshape(V * V, V)
    sy = oh_ref[0, 0]
    sz = oh_ref[0, 1]
    b1 = jax.lax.dot(a, sy, precision=_PREC,
                     preferred_element_type=jnp.float32)
    b3 = jnp.transpose(b1.reshape(V, V, G), (0, 2, 1))
    sc2_ref[...] = b3.reshape(V * G, V)

    def body(x, carry):
        ix = idx_ref[b, x]
        a3_ref[pl.ds(G * x, G)] = sc2_ref[pl.ds(G * ix, G)]
        return carry

    lax.fori_loop(0, G, body, 0)
    out = jax.lax.dot(a3_ref[...], sz, precision=_PREC,
                      preferred_element_type=jnp.float32)
    out_ref[0, 0] = out.reshape(G, G, G)


def kernel(heatmap, vmin_s1, vmax, vmin):
    B, J, V = heatmap.shape[0], heatmap.shape[1], heatmap.shape[2]
    G = 2 * V
    t = jnp.linspace(0.0, 1.0, G).astype(jnp.float32)
    t_row = jnp.broadcast_to(t[None, :], (8, G))
    vb = jnp.concatenate(
        [vmin[:, 0, :], vmax[:, 0, :], jnp.zeros((B, 2), jnp.float32)], axis=1)
    h_flat = heatmap.reshape(B, J, V, V * V)

    oh, idxr, bd = pl.pallas_call(
        functools.partial(_stats_kernel, J=J, V=V, G=G),
        grid=(B,),
        in_specs=[
            pl.BlockSpec((1, J, V, V * V), lambda b: (b, 0, 0, 0)),
            pl.BlockSpec((8, G), lambda b: (0, 0)),
            pl.BlockSpec(memory_space=pltpu.SMEM),
        ],
        out_specs=[
            pl.BlockSpec((1, 2, V, G), lambda b: (b, 0, 0, 0)),
            pl.BlockSpec((1, 8, G), lambda b: (b, 0, 0)),
            pl.BlockSpec((1, 8, 128), lambda b: (b, 0, 0)),
        ],
        out_shape=[
            jax.ShapeDtypeStruct((B, 2, V, G), jnp.float32),
            jax.ShapeDtypeStruct((B, 8, G), jnp.float32),
            jax.ShapeDtypeStruct((B, 8, 128), jnp.float32),
        ],
    )(h_flat, t_row, vb)

    idx_x = idxr[:, 0, :].astype(jnp.int32)

    interp = pl.pallas_call(
        functools.partial(_expand_kernel, V=V, G=G),
        grid=(B, J),
        in_specs=[
            pl.BlockSpec((1, 1, V, V, V), lambda b, j: (b, j, 0, 0, 0)),
            pl.BlockSpec((1, 2, V, G), lambda b, j: (b, 0, 0, 0)),
            pl.BlockSpec(memory_space=pltpu.SMEM),
        ],
        out_specs=pl.BlockSpec((1, 1, G, G, G), lambda b, j: (b, j, 0, 0, 0)),
        out_shape=jax.ShapeDtypeStruct((B, J, G, G, G), jnp.float32),
        scratch_shapes=[
            pltpu.VMEM((V * G, V), jnp.float32),
            pltpu.VMEM((G * G, V), jnp.float32),
        ],
    )(heatmap, oh, idx_x)

    min_b = bd[:, 0, 0:3].reshape(B, 1, 3)
    max_b = bd[:, 0, 3:6].reshape(B, 1, 3)
    return interp, max_b, min_b


# speedup vs baseline: 2.9572x; 2.9572x over previous
import functools

import jax
import jax.numpy as jnp
from jax import lax
from jax.experimental import pallas as pl
from jax.experimental.pallas import tpu as pltpu

BOUND_OFF = 3.0
_PREC = jax.lax.Precision.DEFAULT
_RND = 12582912.0


def _fused_kernel(h_ref, t_ref, vb_ref, ts_ref, out_ref, bd_ref,
                  oh_s, ix_s, bs_s, *, J, V, G, JB):
    b = pl.program_id(0)
    k = pl.program_id(1)
    fmax = float(V - 1)

    @pl.when(k == 0)
    def _stats():
        h = h_ref[0]
        ii = lax.broadcasted_iota(jnp.int32, (J, V, V, V), 1).astype(jnp.float32)
        jj = lax.broadcasted_iota(jnp.int32, (J, V, V, V), 2).astype(jnp.float32)
        kk = lax.broadcasted_iota(jnp.int32, (J, V, V, V), 3).astype(jnp.float32)
        mx = jnp.max(h, axis=(2, 3))
        mx = jnp.max(mx, axis=1).reshape(J, 1, 1, 1)
        mask = (h == mx).astype(jnp.float32)

        def _red(v):
            return jnp.sum(jnp.sum(v, axis=(2, 3)), axis=1)

        cnt = _red(mask)
        pxv = _red(mask * ii) / cnt
        pyv = _red(mask * jj) / cnt
        pzv = _red(mask * kk) / cnt

        def _build(axis, pvec):
            mn = jnp.min(pvec)
            mxp = jnp.max(pvec)
            max_bd = jnp.clip(mxp + BOUND_OFF, 0.0, fmax)
            min_bd = jnp.clip(mn - BOUND_OFF, 0.0, fmax)
            vmin_a = vb_ref[b, axis]
            vmax_a = vb_ref[b, 3 + axis]
            interval = (vmax_a - vmin_a) / fmax
            max_b = vmin_a + max_bd / fmax * (vmax_a - vmin_a)
            min_b = vmin_a + min_bd / fmax * (vmax_a - vmin_a)
            return min_b, max_b, vmin_a, interval

        mnx, mxx, vminx, intx = _build(0, pxv)
        mny, mxy, vminy, inty = _build(1, pyv)
        mnz, mxz, vminz, intz = _build(2, pzv)

        t32 = jnp.broadcast_to(t_ref[0:1, :], (V, G))
        vlane = lax.broadcasted_iota(jnp.int32, (V, G), 0).astype(jnp.float32)
        for slot, (mn, mxw, vmin_a, inter) in enumerate(
            [(mny, mxy, vminy, inty), (mnz, mxz, vminz, intz)]
        ):
            g = mn + t32 * (mxw - mn)
            vox = (g - vmin_a) / inter
            idxf = jnp.clip(jnp.round(vox), 0.0, fmax)
            oh_s[slot] = (idxf == vlane).astype(jnp.float32)

        for x in range(G):
            gx = mnx + ts_ref[0, x] * (mxx - mnx)
            vox = (gx - vminx) / intx
            rnd = jnp.round(vox)
            ix_s[0, x] = jnp.clip(rnd, 0.0, fmax).astype(jnp.int32)

        for p, val in enumerate([mnx, mny, mnz, mxx, mxy, mxz]):
            bs_s[0, p] = val

    lane = lax.broadcasted_iota(jnp.int32, (8, 128), 1)
    bd = jnp.zeros((8, 128), jnp.float32)
    for p in range(6):
        bd = jnp.where(lane == p, bs_s[0, p], bd)
    bd_ref[0] = bd

    sy = oh_s[0]
    sz = oh_s[1]

    def _expand(sc2_ref, a3_ref):
        for jj in range(JB):
            hv = h_ref[0, k * JB + jj]
            ht = jnp.transpose(hv, (0, 2, 1))
            a = ht.reshape(V * V, V)
            b1 = jax.lax.dot(a, sy, precision=_PREC,
                             preferred_element_type=jnp.float32)
            b3 = jnp.transpose(b1.reshape(V, V, G), (0, 2, 1))
            sc2_ref[jj] = b3.reshape(V * G, V)

        ixs = [ix_s[0, x] for x in range(G)]
        for jj in range(JB):
            for x in range(G):
                a3_ref[jj, G * x:G * (x + 1)] = sc2_ref[jj, pl.ds(G * ixs[x], G)]

        for jj in range(JB):
            out = jax.lax.dot(a3_ref[jj], sz, precision=_PREC,
                              preferred_element_type=jnp.float32)
            out_ref[0, jj] = out.reshape(G, G, G)

    pl.run_scoped(
        _expand,
        pltpu.VMEM((JB, V * G, V), jnp.float32),
        pltpu.VMEM((JB, G * G, V), jnp.float32),
    )


def kernel(heatmap, vmin_s1, vmax, vmin):
    B, J, V = heatmap.shape[0], heatmap.shape[1], heatmap.shape[2]
    G = 2 * V
    JB = 3
    t = jnp.linspace(0.0, 1.0, G).astype(jnp.float32)
    t_row = jnp.broadcast_to(t[None, :], (8, G))
    t_s = t[None, :]
    vb = jnp.concatenate(
        [vmin[:, 0, :], vmax[:, 0, :], jnp.zeros((B, 2), jnp.float32)], axis=1)

    interp, bd = pl.pallas_call(
        functools.partial(_fused_kernel, J=J, V=V, G=G, JB=JB),
        grid=(B, J // JB),
        in_specs=[
            pl.BlockSpec((1, J, V, V, V), lambda b, k: (b, 0, 0, 0, 0)),
            pl.BlockSpec((8, G), lambda b, k: (0, 0)),
            pl.BlockSpec(memory_space=pltpu.SMEM),
            pl.BlockSpec(memory_space=pltpu.SMEM),
        ],
        out_specs=[
            pl.BlockSpec((1, JB, G, G, G), lambda b, k: (b, k, 0, 0, 0)),
            pl.BlockSpec((1, 8, 128), lambda b, k: (b, 0, 0)),
        ],
        out_shape=[
            jax.ShapeDtypeStruct((B, J, G, G, G), jnp.float32),
            jax.ShapeDtypeStruct((B, 8, 128), jnp.float32),
        ],
        scratch_shapes=[
            pltpu.VMEM((2, V, G), jnp.float32),
            pltpu.SMEM((1, G), jnp.int32),
            pltpu.SMEM((1, 8), jnp.float32),
        ],
    )(heatmap, t_row, vb, t_s)

    min_b = bd[:, 0, 0:3].reshape(B, 1, 3)
    max_b = bd[:, 0, 3:6].reshape(B, 1, 3)
    return interp, max_b, min_b


# speedup vs baseline: 3.0424x; 1.0288x over previous
import functools

import jax
import jax.numpy as jnp
from jax import lax
from jax.experimental import pallas as pl
from jax.experimental.pallas import tpu as pltpu

BOUND_OFF = 3.0
_PREC = jax.lax.Precision.DEFAULT


def _stats_kernel(h_ref, t_ref, vb_ref, oh_ref, idxr_ref, bd_ref, *, J, V, G):
    b = pl.program_id(0)
    fmax = float(V - 1)
    n = V * V * V
    lanes = n // V
    r_i = lax.broadcasted_iota(jnp.int32, (J, V, lanes), 1)
    c_i = lax.broadcasted_iota(jnp.int32, (J, V, lanes), 2)
    ii = r_i.astype(jnp.float32)
    jj = (c_i // V).astype(jnp.float32)
    kk = (c_i % V).astype(jnp.float32)
    h = h_ref[0]
    mx = jnp.max(h, axis=(1, 2), keepdims=True)
    mask = (h == mx).astype(jnp.float32)
    cnt = jnp.sum(mask, axis=(1, 2))
    pxv = jnp.sum(mask * ii, axis=(1, 2)) / cnt
    pyv = jnp.sum(mask * jj, axis=(1, 2)) / cnt
    pzv = jnp.sum(mask * kk, axis=(1, 2)) / cnt

    def _build(axis, pvec):
        mn = jnp.min(pvec)
        mxp = jnp.max(pvec)
        max_bd = jnp.clip(mxp + BOUND_OFF, 0.0, fmax)
        min_bd = jnp.clip(mn - BOUND_OFF, 0.0, fmax)
        vmin_a = vb_ref[b, axis]
        vmax_a = vb_ref[b, 3 + axis]
        interval = (vmax_a - vmin_a) / fmax
        max_b = vmin_a + max_bd / fmax * (vmax_a - vmin_a)
        min_b = vmin_a + min_bd / fmax * (vmax_a - vmin_a)
        return min_b, max_b, vmin_a, interval

    mnx, mxx, vminx, intx = _build(0, pxv)
    mny, mxy, vminy, inty = _build(1, pyv)
    mnz, mxz, vminz, intz = _build(2, pzv)

    t32 = jnp.broadcast_to(t_ref[0:1, :], (V, G))
    vlane = lax.broadcasted_iota(jnp.int32, (V, G), 0).astype(jnp.float32)
    for slot, (mn, mxw, vmin_a, inter) in enumerate(
        [(mny, mxy, vminy, inty), (mnz, mxz, vminz, intz)]
    ):
        g = mn + t32 * (mxw - mn)
        vox = (g - vmin_a) / inter
        idxf = jnp.clip(jnp.round(vox), 0.0, fmax)
        oh_ref[0, slot] = (idxf == vlane).astype(jnp.float32)

    gx = mnx + t_ref[...] * (mxx - mnx)
    voxx = (gx - vminx) / intx
    idxr_ref[0] = jnp.clip(jnp.round(voxx), 0.0, fmax)

    lane = lax.broadcasted_iota(jnp.int32, (8, 128), 1)
    bd = jnp.zeros((8, 128), jnp.float32)
    for p, val in enumerate([mnx, mny, mnz, mxx, mxy, mxz]):
        bd = jnp.where(lane == p, val, bd)
    bd_ref[0] = bd


def _expand_kernel(h_ref, oh_ref, idx_ref, out_ref, sc2_ref, a3_ref, *, V, G, JB):
    b = pl.program_id(0)
    sy = oh_ref[0, 0]
    sz = oh_ref[0, 1]
    for jj in range(JB):
        hv = h_ref[0, jj]
        ht = jnp.transpose(hv, (0, 2, 1))
        a = ht.reshape(V * V, V)
        b1 = jax.lax.dot(a, sy, precision=_PREC,
                         preferred_element_type=jnp.float32)
        b3 = jnp.transpose(b1.reshape(V, V, G), (0, 2, 1))
        sc2_ref[jj] = b3.reshape(V * G, V)

    ixs = [idx_ref[b, x] for x in range(G)]
    for jj in range(JB):
        for x in range(G):
            a3_ref[jj, G * x:G * (x + 1)] = sc2_ref[jj, pl.ds(G * ixs[x], G)]

    for jj in range(JB):
        out = jax.lax.dot(a3_ref[jj], sz, precision=_PREC,
                          preferred_element_type=jnp.float32)
        out_ref[0, jj] = out.reshape(G, G, G)


def kernel(heatmap, vmin_s1, vmax, vmin):
    B, J, V = heatmap.shape[0], heatmap.shape[1], heatmap.shape[2]
    G = 2 * V
    t = jnp.linspace(0.0, 1.0, G).astype(jnp.float32)
    t_row = jnp.broadcast_to(t[None, :], (8, G))
    vb = jnp.concatenate(
        [vmin[:, 0, :], vmax[:, 0, :], jnp.zeros((B, 2), jnp.float32)], axis=1)
    h_flat = heatmap.reshape(B, J, V, V * V)

    oh, idxr, bd = pl.pallas_call(
        functools.partial(_stats_kernel, J=J, V=V, G=G),
        grid=(B,),
        in_specs=[
            pl.BlockSpec((1, J, V, V * V), lambda b: (b, 0, 0, 0)),
            pl.BlockSpec((8, G), lambda b: (0, 0)),
            pl.BlockSpec(memory_space=pltpu.SMEM),
        ],
        out_specs=[
            pl.BlockSpec((1, 2, V, G), lambda b: (b, 0, 0, 0)),
            pl.BlockSpec((1, 8, G), lambda b: (b, 0, 0)),
            pl.BlockSpec((1, 8, 128), lambda b: (b, 0, 0)),
        ],
        out_shape=[
            jax.ShapeDtypeStruct((B, 2, V, G), jnp.float32),
            jax.ShapeDtypeStruct((B, 8, G), jnp.float32),
            jax.ShapeDtypeStruct((B, 8, 128), jnp.float32),
        ],
    )(h_flat, t_row, vb)

    idx_x = idxr[:, 0, :].astype(jnp.int32)

    JB = 3
    interp = pl.pallas_call(
        functools.partial(_expand_kernel, V=V, G=G, JB=JB),
        grid=(B, J // JB),
        in_specs=[
            pl.BlockSpec((1, JB, V, V, V), lambda b, j: (b, j, 0, 0, 0)),
            pl.BlockSpec((1, 2, V, G), lambda b, j: (b, 0, 0, 0)),
            pl.BlockSpec(memory_space=pltpu.SMEM),
        ],
        out_specs=pl.BlockSpec((1, JB, G, G, G), lambda b, j: (b, j, 0, 0, 0)),
        out_shape=jax.ShapeDtypeStruct((B, J, G, G, G), jnp.float32),
        scratch_shapes=[
            pltpu.VMEM((JB, V * G, V), jnp.float32),
            pltpu.VMEM((JB, G * G, V), jnp.float32),
        ],
    )(heatmap, oh, idx_x)

    min_b = bd[:, 0, 0:3].reshape(B, 1, 3)
    max_b = bd[:, 0, 3:6].reshape(B, 1, 3)
    return interp, max_b, min_b
